# Initial kernel scaffold; baseline (speedup 1.0000x reference)
import jax
import jax.numpy as jnp
from jax.experimental import pallas as pl


def kernel(h, batch, cdr_mask, iface_mask, Wk, Wv, Wq, Wres, Wout, ln_kv_g, ln_kv_b, ln_q_g, ln_q_b, cdr_bias, iface_bias, logit_scale):
    raise NotImplementedError("write your pallas kernel here")


# speedup vs baseline: 12.1028x; 12.1028x over previous
import functools

import jax
import jax.numpy as jnp
import numpy as np
from jax import lax
from jax.experimental import pallas as pl
from jax.experimental.pallas import tpu as pltpu
from jax.experimental.pallas import tpu_sc as plsc

N = 32768
D = 128
H = 4
HD = D // H
B = 16
MRS = 0.2

NC = 2
NS = 16
NW = NC * NS
L = 16
NVD = D // L
TPW = N // NW
CW = 128
NCH = TPW // CW
NG = CW // L
F32 = jnp.float32
NEG = -1e30

_mesh = plsc.VectorSubcoreMesh(
    core_axis_name="c", subcore_axis_name="s", num_cores=NC, num_subcores=NS)


def _wid():
    return lax.axis_index("s") * NC + lax.axis_index("c")


def _splat(x):
    return jnp.full((L,), x, F32)


def _lane():
    return lax.broadcasted_iota(jnp.int32, (L,), 0)


_GDN = lax.GatherDimensionNumbers(
    offset_dims=(), collapsed_slice_dims=(0,), start_index_map=(0,))


def _gather(v, idx):
    return lax.gather(v, idx[:, None], _GDN, slice_sizes=(1,),
                      mode=lax.GatherScatterMode.PROMISE_IN_BOUNDS)


def _take(v, i):
    return _gather(v, jnp.full((L,), i, jnp.int32))


def _allsum(v):
    lane = _lane()
    for k in (8, 4, 2, 1):
        v = v + _gather(v, lane ^ k)
    return v


def _rsqrt_v(x):
    i = lax.bitcast_convert_type(x, jnp.int32)
    i = jnp.full((L,), 0x5F3759DF, jnp.int32) - (i >> 1)
    y = lax.bitcast_convert_type(i, F32)
    for _ in range(3):
        y = y * (1.5 - 0.5 * x * y * y)
    return y


def _k1_body(h_hbm, batch_hbm,
             psum_hbm, pmax_hbm, pcnt_hbm, mu_hbm, rs_hbm,
             hbuf, batchbuf, accsum, accmax, acccnt, mubuf, rsbuf):
    w = _wid()
    tok0 = w * TPW
    pltpu.sync_copy(batch_hbm.at[pl.ds(tok0, TPW)], batchbuf)
    lane = _lane()

    @pl.loop(0, B * NVD)
    def _(i):
        accsum[pl.ds(i * L, L)] = jnp.zeros((L,), F32)
        accmax[pl.ds(i * L, L)] = jnp.full((L,), NEG, F32)

    @pl.loop(0, B)
    def _(i):
        acccnt[pl.ds(i * L, L)] = jnp.zeros((L,), F32)

    @pl.loop(0, NCH)
    def _(ch):
        pltpu.sync_copy(h_hbm.at[pl.ds((tok0 + ch * CW) * D, CW * D)], hbuf)

        @pl.loop(0, NG)
        def _(g):
            gt = ch * NG + g
            batchrow = batchbuf[pl.ds(gt * L, L)]
            murow = jnp.zeros((L,), F32)
            rsrow = jnp.zeros((L,), F32)
            for i in range(L):
                off = (g * L + i) * D
                hv = [hbuf[pl.ds(off + j * L, L)] for j in range(NVD)]
                s = hv[0]
                sq = hv[0] * hv[0]
                for j in range(1, NVD):
                    s = s + hv[j]
                    sq = hv[j] * hv[j] + sq
                mu = _allsum(s) * (1.0 / D)
                var = _allsum(sq) * (1.0 / D) - mu * mu
                rs = _rsqrt_v(var + 1e-5)
                murow = jnp.where(lane == i, mu, murow)
                rsrow = jnp.where(lane == i, rs, rsrow)
                seg = batchrow[i]
                sbase = seg * D
                for j in range(NVD):
                    idx = pl.ds(sbase + j * L, L)
                    accsum[idx] = accsum[idx] + hv[j]
                    accmax[idx] = jnp.maximum(accmax[idx], hv[j])
                ci = pl.ds(seg * L, L)
                acccnt[ci] = acccnt[ci] + 1.0
            mubuf[pl.ds(gt * L, L)] = murow
            rsbuf[pl.ds(gt * L, L)] = rsrow

    pltpu.sync_copy(accsum, psum_hbm.at[pl.ds(w * (B * D), B * D)])
    pltpu.sync_copy(accmax, pmax_hbm.at[pl.ds(w * (B * D), B * D)])
    pltpu.sync_copy(acccnt, pcnt_hbm.at[pl.ds(w * (B * L), B * L)])
    pltpu.sync_copy(mubuf, mu_hbm.at[pl.ds(tok0, TPW)])
    pltpu.sync_copy(rsbuf, rs_hbm.at[pl.ds(tok0, TPW)])


_k1 = functools.partial(
    pl.kernel,
    out_type=(
        jax.ShapeDtypeStruct((NW * B * D,), F32),
        jax.ShapeDtypeStruct((NW * B * D,), F32),
        jax.ShapeDtypeStruct((NW * B * L,), F32),
        jax.ShapeDtypeStruct((N,), F32),
        jax.ShapeDtypeStruct((N,), F32),
    ),
    mesh=_mesh,
    scratch_types=[
        pltpu.VMEM((CW * D,), F32),
        pltpu.VMEM((TPW,), jnp.int32),
        pltpu.VMEM((B * D,), F32),
        pltpu.VMEM((B * D,), F32),
        pltpu.VMEM((B * L,), F32),
        pltpu.VMEM((TPW,), F32),
        pltpu.VMEM((TPW,), F32),
    ],
)(_k1_body)


def _k2_body(psum_hbm, pmax_hbm, pcnt_hbm, wqt_hbm, wk_hbm,
             lnqg_hbm, lnqb_hbm, lnkg_hbm, lnkb_hbm, params_hbm,
             mean_hbm, qpg_hbm, s1c_hbm,
             colbuf, qinbuf, qnbuf, wqbuf, wkbuf, qpgstage, s1cstage,
             pcntbuf, lnqgbuf, lnqbbuf, lnkgbuf, lnkbbuf, paramsbuf):
    w = _wid()

    @pl.when(w < B)
    def _():
        b = w
        pltpu.sync_copy(wqt_hbm, wqbuf)
        pltpu.sync_copy(wk_hbm, wkbuf)
        pltpu.sync_copy(pcnt_hbm, pcntbuf)
        pltpu.sync_copy(lnqg_hbm, lnqgbuf)
        pltpu.sync_copy(lnqb_hbm, lnqbbuf)
        pltpu.sync_copy(lnkg_hbm, lnkgbuf)
        pltpu.sync_copy(lnkb_hbm, lnkbbuf)
        pltpu.sync_copy(params_hbm, paramsbuf)

        for wi in range(NW):
            pltpu.sync_copy(psum_hbm.at[pl.ds(wi * (B * D) + b * D, D)],
                            colbuf.at[pl.ds(wi * D, D)])
        sums = []
        for j in range(NVD):
            a = colbuf[pl.ds(j * L, L)]
            for wi in range(1, NW):
                a = a + colbuf[pl.ds(wi * D + j * L, L)]
            sums.append(a)
        for wi in range(NW):
            pltpu.sync_copy(pmax_hbm.at[pl.ds(wi * (B * D) + b * D, D)],
                            colbuf.at[pl.ds(wi * D, D)])
        maxs = []
        for j in range(NVD):
            a = colbuf[pl.ds(j * L, L)]
            for wi in range(1, NW):
                a = jnp.maximum(a, colbuf[pl.ds(wi * D + j * L, L)])
            maxs.append(a)
        cntv = pcntbuf[pl.ds(b * L, L)]
        for wi in range(1, NW):
            cntv = cntv + pcntbuf[pl.ds(wi * (B * L) + b * L, L)]
        for j in range(NVD):
            qinbuf[pl.ds(j * L, L)] = sums[j] / cntv
            qinbuf[pl.ds(D + j * L, L)] = maxs[j]
        pltpu.sync_copy(qinbuf.at[pl.ds(0, D)], mean_hbm.at[pl.ds(b * D, D)])

        def qstep(gi, accs):
            qrow = qinbuf[pl.ds(gi * L, L)]
            for i in range(L):
                sp = _take(qrow, i)
                accs = tuple(
                    accs[j] + sp * wqbuf[pl.ds((gi * L + i) * D + j * L, L)]
                    for j in range(NVD))
            return accs

        accs = lax.fori_loop(0, 2 * D // L, qstep,
                             tuple(jnp.zeros((L,), F32) for _ in range(NVD)))
        s = accs[0]
        sq = accs[0] * accs[0]
        for j in range(1, NVD):
            s = s + accs[j]
            sq = accs[j] * accs[j] + sq
        musp = _allsum(s) * (1.0 / D)
        var = _allsum(sq) * (1.0 / D) - musp * musp
        rssp = _rsqrt_v(var + 1e-5)
        for j in range(NVD):
            qn = ((accs[j] - musp) * rssp * lnqgbuf[pl.ds(j * L, L)]
                  + lnqbbuf[pl.ds(j * L, L)])
            qnbuf[pl.ds(j * L, L)] = qn

        prow = paramsbuf[pl.ds(0, L)]
        ssp = _take(prow, 2) * np.float32(1.0 / np.sqrt(HD))
        lane = _lane()
        s1c = jnp.zeros((L,), F32)
        for hh in range(H):
            def pstep(g2, accs, hh=hh):
                qnrow = qnbuf[pl.ds(hh * HD + g2 * L, L)]
                for i in range(L):
                    r = hh * HD + g2 * L + i
                    sp = _take(qnrow, i)
                    accs = tuple(
                        accs[j] + sp * wkbuf[pl.ds(r * D + j * L, L)]
                        for j in range(NVD))
                return accs

            paccs = lax.fori_loop(0, HD // L, pstep,
                                  tuple(jnp.zeros((L,), F32)
                                        for _ in range(NVD)))
            qp = [p * ssp for p in paccs]
            s1v = None
            ccv = None
            for j in range(NVD):
                qpg = qp[j] * lnkgbuf[pl.ds(j * L, L)]
                qpgstage[pl.ds(hh * D + j * L, L)] = qpg
                cterm = qp[j] * lnkbbuf[pl.ds(j * L, L)]
                s1v = qpg if s1v is None else s1v + qpg
                ccv = cterm if ccv is None else ccv + cterm
            s1c = jnp.where(lane == hh, _allsum(s1v), s1c)
            s1c = jnp.where(lane == H + hh, _allsum(ccv), s1c)
        s1cstage[pl.ds(0, L)] = s1c
        pltpu.sync_copy(qpgstage, qpg_hbm.at[pl.ds(b * (H * D), H * D)])
        pltpu.sync_copy(s1cstage, s1c_hbm.at[pl.ds(b * L, L)])


_k2 = functools.partial(
    pl.kernel,
    out_type=(
        jax.ShapeDtypeStruct((B * D,), F32),
        jax.ShapeDtypeStruct((B * H * D,), F32),
        jax.ShapeDtypeStruct((B * L,), F32),
    ),
    mesh=_mesh,
    scratch_types=[
        pltpu.VMEM((NW * D,), F32),
        pltpu.VMEM((2 * D,), F32),
        pltpu.VMEM((D,), F32),
        pltpu.VMEM((2 * D * D,), F32),
        pltpu.VMEM((D * D,), F32),
        pltpu.VMEM((H * D,), F32),
        pltpu.VMEM((L,), F32),
        pltpu.VMEM((NW * B * L,), F32),
        pltpu.VMEM((D,), F32),
        pltpu.VMEM((D,), F32),
        pltpu.VMEM((D,), F32),
        pltpu.VMEM((D,), F32),
        pltpu.VMEM((L,), F32),
    ],
)(_k2_body)


def _k3_body(h_hbm, batch_hbm, mu_hbm, rs_hbm, qpg_hbm, s1c_hbm,
             cdr_hbm, iface_hbm, params_hbm,
             log_hbm, pmaxl_hbm,
             hbuf, batchbuf, mubuf, rsbuf, cdbuf, ifbuf, biasbuf,
             qpgbuf, s1cbuf, paramsbuf, logbuf, lmax):
    w = _wid()
    tok0 = w * TPW
    pltpu.sync_copy(batch_hbm.at[pl.ds(tok0, TPW)], batchbuf)
    pltpu.sync_copy(mu_hbm.at[pl.ds(tok0, TPW)], mubuf)
    pltpu.sync_copy(rs_hbm.at[pl.ds(tok0, TPW)], rsbuf)
    pltpu.sync_copy(cdr_hbm.at[pl.ds(tok0, TPW)], cdbuf)
    pltpu.sync_copy(iface_hbm.at[pl.ds(tok0, TPW)], ifbuf)
    pltpu.sync_copy(qpg_hbm, qpgbuf)
    pltpu.sync_copy(s1c_hbm, s1cbuf)
    pltpu.sync_copy(params_hbm, paramsbuf)

    prow = paramsbuf[pl.ds(0, L)]
    cbsp = _take(prow, 0)
    ibsp = _take(prow, 1)
    lane = _lane()

    @pl.loop(0, TPW // L)
    def _(i):
        idx = pl.ds(i * L, L)
        biasbuf[idx] = cbsp * cdbuf[idx] + ibsp * ifbuf[idx]

    @pl.loop(0, B)
    def _(i):
        lmax[pl.ds(i * L, L)] = jnp.full((L,), NEG, F32)

    @pl.loop(0, NCH)
    def _(ch):
        pltpu.sync_copy(h_hbm.at[pl.ds((tok0 + ch * CW) * D, CW * D)], hbuf)

        @pl.loop(0, NG)
        def _(g):
            gt = ch * NG + g
            batchrow = batchbuf[pl.ds(gt * L, L)]
            murow = mubuf[pl.ds(gt * L, L)]
            rsrow = rsbuf[pl.ds(gt * L, L)]
            biasrow = biasbuf[pl.ds(gt * L, L)]
            rmrow = rsrow * murow
            for i in range(L):
                off = (g * L + i) * D
                hv = [hbuf[pl.ds(off + j * L, L)] for j in range(NVD)]
                seg = batchrow[i]
                qb = seg * (H * D)
                rssp = _take(rsrow, i)
                s1crow = s1cbuf[pl.ds(seg * L, L)]
                lrow = jnp.full((L,), NEG, F32)
                for hh in range(H):
                    p = hv[0] * qpgbuf[pl.ds(qb + hh * D, L)]
                    for j in range(1, NVD):
                        p = hv[j] * qpgbuf[pl.ds(qb + hh * D + j * L, L)] + p
                    lrow = jnp.where(lane == hh, rssp * _allsum(p), lrow)
                s1l = _gather(s1crow, lane & 3)
                cl = _gather(s1crow, (lane & 3) + H)
                lrow = (lrow - _take(rmrow, i) * s1l + cl
                        + _take(biasrow, i))
                logbuf[pl.ds((gt * L + i) * L, L)] = lrow
                mi = pl.ds(seg * L, L)
                lmax[mi] = jnp.maximum(lmax[mi], lrow)

    pltpu.sync_copy(logbuf, log_hbm.at[pl.ds(tok0 * L, TPW * L)])
    pltpu.sync_copy(lmax, pmaxl_hbm.at[pl.ds(w * (B * L), B * L)])


_k3 = functools.partial(
    pl.kernel,
    out_type=(
        jax.ShapeDtypeStruct((N * L,), F32),
        jax.ShapeDtypeStruct((NW * B * L,), F32),
    ),
    mesh=_mesh,
    scratch_types=[
        pltpu.VMEM((CW * D,), F32),
        pltpu.VMEM((TPW,), jnp.int32),
        pltpu.VMEM((TPW,), F32),
        pltpu.VMEM((TPW,), F32),
        pltpu.VMEM((TPW,), F32),
        pltpu.VMEM((TPW,), F32),
        pltpu.VMEM((TPW,), F32),
        pltpu.VMEM((B * H * D,), F32),
        pltpu.VMEM((B * L,), F32),
        pltpu.VMEM((L,), F32),
        pltpu.VMEM((TPW * L,), F32),
        pltpu.VMEM((B * L,), F32),
    ],
)(_k3_body)


def _k4_body(h_hbm, batch_hbm, mu_hbm, rs_hbm, log_hbm, pmaxl_hbm,
             pp_hbm, sp_hbm, t1_hbm,
             hbuf, batchbuf, mubuf, rsbuf, logbuf, pmaxlbuf, mbuf,
             pacc, sacc, t1acc):
    w = _wid()
    tok0 = w * TPW
    pltpu.sync_copy(batch_hbm.at[pl.ds(tok0, TPW)], batchbuf)
    pltpu.sync_copy(mu_hbm.at[pl.ds(tok0, TPW)], mubuf)
    pltpu.sync_copy(rs_hbm.at[pl.ds(tok0, TPW)], rsbuf)
    pltpu.sync_copy(pmaxl_hbm, pmaxlbuf)

    @pl.loop(0, B)
    def _(i):
        a = pmaxlbuf[pl.ds(i * L, L)]
        for wi in range(1, NW):
            a = jnp.maximum(a, pmaxlbuf[pl.ds(wi * (B * L) + i * L, L)])
        mbuf[pl.ds(i * L, L)] = a

    @pl.loop(0, B * H * D // L)
    def _(i):
        pacc[pl.ds(i * L, L)] = jnp.zeros((L,), F32)

    @pl.loop(0, B)
    def _(i):
        sacc[pl.ds(i * L, L)] = jnp.zeros((L,), F32)
        t1acc[pl.ds(i * L, L)] = jnp.zeros((L,), F32)

    @pl.loop(0, NCH)
    def _(ch):
        pltpu.sync_copy(h_hbm.at[pl.ds((tok0 + ch * CW) * D, CW * D)], hbuf)
        pltpu.sync_copy(log_hbm.at[pl.ds((tok0 + ch * CW) * L, CW * L)],
                        logbuf)

        @pl.loop(0, NG)
        def _(g):
            gt = ch * NG + g
            batchrow = batchbuf[pl.ds(gt * L, L)]
            murow = mubuf[pl.ds(gt * L, L)]
            rsrow = rsbuf[pl.ds(gt * L, L)]
            rmrow = rsrow * murow
            for i in range(L):
                off = (g * L + i) * D
                hv = [hbuf[pl.ds(off + j * L, L)] for j in range(NVD)]
                seg = batchrow[i]
                rmsp = _take(rmrow, i)
                rssp = _take(rsrow, i)
                lrow = logbuf[pl.ds((g * L + i) * L, L)]
                mrow = mbuf[pl.ds(seg * L, L)]
                erow = jnp.exp(lrow - mrow)
                si = pl.ds(seg * L, L)
                sacc[si] = sacc[si] + erow
                t1acc[si] = erow * rmsp + t1acc[si]
                for hh in range(H):
                    wr = _take(erow, hh) * rssp
                    pb = seg * (H * D) + hh * D
                    for j in range(NVD):
                        idx = pl.ds(pb + j * L, L)
                        pacc[idx] = wr * hv[j] + pacc[idx]

    pltpu.sync_copy(pacc, pp_hbm.at[pl.ds(w * (B * H * D), B * H * D)])
    pltpu.sync_copy(sacc, sp_hbm.at[pl.ds(w * (B * L), B * L)])
    pltpu.sync_copy(t1acc, t1_hbm.at[pl.ds(w * (B * L), B * L)])


_k4 = functools.partial(
    pl.kernel,
    out_type=(
        jax.ShapeDtypeStruct((NW * B * H * D,), F32),
        jax.ShapeDtypeStruct((NW * B * L,), F32),
        jax.ShapeDtypeStruct((NW * B * L,), F32),
    ),
    mesh=_mesh,
    scratch_types=[
        pltpu.VMEM((CW * D,), F32),
        pltpu.VMEM((TPW,), jnp.int32),
        pltpu.VMEM((TPW,), F32),
        pltpu.VMEM((TPW,), F32),
        pltpu.VMEM((CW * L,), F32),
        pltpu.VMEM((NW * B * L,), F32),
        pltpu.VMEM((B * L,), F32),
        pltpu.VMEM((B * H * D,), F32),
        pltpu.VMEM((B * L,), F32),
        pltpu.VMEM((B * L,), F32),
    ],
)(_k4_body)


def _k5_body(pp_hbm, sp_hbm, t1_hbm, mean_hbm, wv_hbm, woutt_hbm, wrest_hbm,
             lnkg_hbm, lnkb_hbm,
             out_hbm,
             pbuf, sbuf, t1buf, meanbuf, wvbuf, woutbuf, wresbuf,
             gkbuf, bkbuf, abuf, obuf):
    w = _wid()
    b = w // 2
    o0 = (w % 2) * (D // 2)
    for wi in range(NW):
        pltpu.sync_copy(pp_hbm.at[pl.ds(wi * (B * H * D) + b * (H * D), H * D)],
                        pbuf.at[pl.ds(wi * (H * D), H * D)])
    pltpu.sync_copy(sp_hbm, sbuf)
    pltpu.sync_copy(t1_hbm, t1buf)
    pltpu.sync_copy(mean_hbm.at[pl.ds(b * D, D)], meanbuf)
    pltpu.sync_copy(wv_hbm, wvbuf)
    pltpu.sync_copy(woutt_hbm, woutbuf)
    pltpu.sync_copy(wrest_hbm, wresbuf)
    pltpu.sync_copy(lnkg_hbm, gkbuf)
    pltpu.sync_copy(lnkb_hbm, bkbuf)

    srow = sbuf[pl.ds(b * L, L)]
    t1row = t1buf[pl.ds(b * L, L)]
    for wi in range(1, NW):
        srow = srow + sbuf[pl.ds(wi * (B * L) + b * L, L)]
        t1row = t1row + t1buf[pl.ds(wi * (B * L) + b * L, L)]

    for hh in range(H):
        sv = _take(srow, hh)
        t1v = _take(t1row, hh)
        for j in range(NVD):
            pm = pbuf[pl.ds(hh * D + j * L, L)]
            for wi in range(1, NW):
                pm = pm + pbuf[pl.ds(wi * (H * D) + hh * D + j * L, L)]
            gk = gkbuf[pl.ds(j * L, L)]
            bk = bkbuf[pl.ds(j * L, L)]
            abuf[pl.ds(hh * D + j * L, L)] = (gk * (pm - t1v) + bk * sv) / sv

    zero4 = tuple(jnp.zeros((L,), F32) for _ in range(4))
    oacc = zero4
    for hh in range(H):
        av = [abuf[pl.ds(hh * D + j * L, L)] for j in range(NVD)]

        def kstep(k2, oaccs, hh=hh, av=av):
            k = hh * HD + k2
            p = av[0] * wvbuf[pl.ds(k * D, L)]
            for j in range(1, NVD):
                p = av[j] * wvbuf[pl.ds(k * D + j * L, L)] + p
            gs = _allsum(p)
            return tuple(oaccs[m] + gs * woutbuf[pl.ds(k * D + o0 + m * L, L)]
                         for m in range(4))

        oacc = lax.fori_loop(0, HD, kstep, oacc)

    def rstep(gk2, raccs):
        mrow = meanbuf[pl.ds(gk2 * L, L)]
        for i in range(L):
            ms = _take(mrow, i)
            k = gk2 * L + i
            raccs = tuple(
                raccs[m] + ms * wresbuf[pl.ds(k * D + o0 + m * L, L)]
                for m in range(4))
        return raccs

    racc = lax.fori_loop(0, D // L, rstep, zero4)
    for m in range(4):
        obuf[pl.ds(m * L, L)] = oacc[m] + np.float32(MRS) * racc[m]
    pltpu.sync_copy(obuf, out_hbm.at[pl.ds(b * D + o0, D // 2)])


_k5 = functools.partial(
    pl.kernel,
    out_type=jax.ShapeDtypeStruct((B * D,), F32),
    mesh=_mesh,
    scratch_types=[
        pltpu.VMEM((NW * H * D,), F32),
        pltpu.VMEM((NW * B * L,), F32),
        pltpu.VMEM((NW * B * L,), F32),
        pltpu.VMEM((D,), F32),
        pltpu.VMEM((D * D,), F32),
        pltpu.VMEM((D * D,), F32),
        pltpu.VMEM((D * D,), F32),
        pltpu.VMEM((D,), F32),
        pltpu.VMEM((D,), F32),
        pltpu.VMEM((H * D,), F32),
        pltpu.VMEM((D // 2,), F32),
    ],
)(_k5_body)


def kernel(h, batch, cdr_mask, iface_mask, Wk, Wv, Wq, Wres, Wout,
           ln_kv_g, ln_kv_b, ln_q_g, ln_q_b, cdr_bias, iface_bias,
           logit_scale):
    h_flat = h.reshape(-1)
    cdrf = cdr_mask.astype(F32)
    ifacef = iface_mask.astype(F32)
    params = jnp.concatenate([
        jnp.stack([cdr_bias.astype(F32), iface_bias.astype(F32),
                   logit_scale.astype(F32)]),
        jnp.zeros((L - 3,), F32),
    ])
    wqt = Wq.T.reshape(-1)
    wk = Wk.reshape(-1)
    wv = Wv.reshape(-1)
    woutt = Wout.T.reshape(-1)
    wrest = Wres.T.reshape(-1)

    psum, pmax, pcnt, mu, rs = _k1(h_flat, batch)
    mean, qpg, s1c = _k2(psum, pmax, pcnt, wqt, wk,
                         ln_q_g, ln_q_b, ln_kv_g, ln_kv_b, params)
    logits, pmaxl = _k3(h_flat, batch, mu, rs, qpg, s1c, cdrf, ifacef, params)
    pp, sp, t1 = _k4(h_flat, batch, mu, rs, logits, pmaxl)
    out = _k5(pp, sp, t1, mean, wv, woutt, wrest, ln_kv_g, ln_kv_b)
    return out.reshape(B, D)


# speedup vs baseline: 13.5956x; 1.1233x over previous
import functools

import jax
import jax.numpy as jnp
import numpy as np
from jax import lax
from jax.experimental import pallas as pl
from jax.experimental.pallas import tpu as pltpu
from jax.experimental.pallas import tpu_sc as plsc

N = 32768
D = 128
H = 4
HD = D // H
B = 16
MRS = 0.2

NC = 2
NS = 16
NW = NC * NS
L = 16
NVD = D // L
TPW = N // NW
CW = 128
NCH = TPW // CW
NG = CW // L
F32 = jnp.float32
NEG = -1e30

_mesh = plsc.VectorSubcoreMesh(
    core_axis_name="c", subcore_axis_name="s", num_cores=NC, num_subcores=NS)


def _wid():
    return lax.axis_index("s") * NC + lax.axis_index("c")


def _splat(x):
    return jnp.full((L,), x, F32)


def _lane():
    return lax.broadcasted_iota(jnp.int32, (L,), 0)


_GDN = lax.GatherDimensionNumbers(
    offset_dims=(), collapsed_slice_dims=(0,), start_index_map=(0,))


def _gather(v, idx):
    return lax.gather(v, idx[:, None], _GDN, slice_sizes=(1,),
                      mode=lax.GatherScatterMode.PROMISE_IN_BOUNDS)


def _take(v, i):
    return _gather(v, jnp.full((L,), i, jnp.int32))


def _allsum(v):
    lane = _lane()
    for k in (8, 4, 2, 1):
        v = v + _gather(v, lane ^ k)
    return v


def _rsqrt_v(x):
    i = lax.bitcast_convert_type(x, jnp.int32)
    i = jnp.full((L,), 0x5F3759DF, jnp.int32) - (i >> 1)
    y = lax.bitcast_convert_type(i, F32)
    for _ in range(3):
        y = y * (1.5 - 0.5 * x * y * y)
    return y


def _k1_body(h_hbm, batch_hbm,
             psum_hbm, pmax_hbm, pcnt_hbm, mu_hbm, rs_hbm,
             hbuf, batchbuf, accsum, accmax, acccnt, mubuf, rsbuf):
    w = _wid()
    tok0 = w * TPW
    pltpu.sync_copy(batch_hbm.at[pl.ds(tok0, TPW)], batchbuf)
    lane = _lane()

    @pl.loop(0, B * NVD)
    def _(i):
        accsum[pl.ds(i * L, L)] = jnp.zeros((L,), F32)
        accmax[pl.ds(i * L, L)] = jnp.full((L,), NEG, F32)

    @pl.loop(0, B)
    def _(i):
        acccnt[pl.ds(i * L, L)] = jnp.zeros((L,), F32)

    @pl.loop(0, NCH)
    def _(ch):
        pltpu.sync_copy(h_hbm.at[pl.ds((tok0 + ch * CW) * D, CW * D)], hbuf)
        c0 = ch * CW
        brow0 = batchbuf[pl.ds(c0, L)]
        brow1 = batchbuf[pl.ds(c0 + CW - L, L)]
        uniform = brow0[0] == brow1[L - 1]

        @pl.when(uniform)
        def _():
            seg = brow0[0]
            sbase = seg * D

            def gloop(g, carry):
                accs = list(carry[:NVD])
                accm = list(carry[NVD:2 * NVD])
                cnt = carry[2 * NVD]
                gt = ch * NG + g
                murow = jnp.zeros((L,), F32)
                rsrow = jnp.zeros((L,), F32)
                for i in range(L):
                    off = (g * L + i) * D
                    hv = [hbuf[pl.ds(off + j * L, L)] for j in range(NVD)]
                    s = hv[0]
                    sq = hv[0] * hv[0]
                    for j in range(1, NVD):
                        s = s + hv[j]
                        sq = hv[j] * hv[j] + sq
                    mu = _allsum(s) * (1.0 / D)
                    var = _allsum(sq) * (1.0 / D) - mu * mu
                    rs = _rsqrt_v(var + 1e-5)
                    murow = jnp.where(lane == i, mu, murow)
                    rsrow = jnp.where(lane == i, rs, rsrow)
                    for j in range(NVD):
                        accs[j] = accs[j] + hv[j]
                        accm[j] = jnp.maximum(accm[j], hv[j])
                mubuf[pl.ds(gt * L, L)] = murow
                rsbuf[pl.ds(gt * L, L)] = rsrow
                return (*accs, *accm, cnt + F32(L))

            init = (tuple(accsum[pl.ds(sbase + j * L, L)] for j in range(NVD))
                    + tuple(accmax[pl.ds(sbase + j * L, L)]
                            for j in range(NVD))
                    + (acccnt[pl.ds(seg * L, L)],))
            res = lax.fori_loop(0, NG, gloop, init)
            for j in range(NVD):
                accsum[pl.ds(sbase + j * L, L)] = res[j]
                accmax[pl.ds(sbase + j * L, L)] = res[NVD + j]
            acccnt[pl.ds(seg * L, L)] = res[2 * NVD]

        @pl.when(jnp.logical_not(uniform))
        def _():
            @pl.loop(0, NG)
            def _(g):
                gt = ch * NG + g
                batchrow = batchbuf[pl.ds(gt * L, L)]
                murow = jnp.zeros((L,), F32)
                rsrow = jnp.zeros((L,), F32)
                for i in range(L):
                    off = (g * L + i) * D
                    hv = [hbuf[pl.ds(off + j * L, L)] for j in range(NVD)]
                    s = hv[0]
                    sq = hv[0] * hv[0]
                    for j in range(1, NVD):
                        s = s + hv[j]
                        sq = hv[j] * hv[j] + sq
                    mu = _allsum(s) * (1.0 / D)
                    var = _allsum(sq) * (1.0 / D) - mu * mu
                    rs = _rsqrt_v(var + 1e-5)
                    murow = jnp.where(lane == i, mu, murow)
                    rsrow = jnp.where(lane == i, rs, rsrow)
                    seg = batchrow[i]
                    sbase = seg * D
                    for j in range(NVD):
                        idx = pl.ds(sbase + j * L, L)
                        accsum[idx] = accsum[idx] + hv[j]
                        accmax[idx] = jnp.maximum(accmax[idx], hv[j])
                    ci = pl.ds(seg * L, L)
                    acccnt[ci] = acccnt[ci] + 1.0
                mubuf[pl.ds(gt * L, L)] = murow
                rsbuf[pl.ds(gt * L, L)] = rsrow

    pltpu.sync_copy(accsum, psum_hbm.at[pl.ds(w * (B * D), B * D)])
    pltpu.sync_copy(accmax, pmax_hbm.at[pl.ds(w * (B * D), B * D)])
    pltpu.sync_copy(acccnt, pcnt_hbm.at[pl.ds(w * (B * L), B * L)])
    pltpu.sync_copy(mubuf, mu_hbm.at[pl.ds(tok0, TPW)])
    pltpu.sync_copy(rsbuf, rs_hbm.at[pl.ds(tok0, TPW)])


_k1 = functools.partial(
    pl.kernel,
    out_type=(
        jax.ShapeDtypeStruct((NW * B * D,), F32),
        jax.ShapeDtypeStruct((NW * B * D,), F32),
        jax.ShapeDtypeStruct((NW * B * L,), F32),
        jax.ShapeDtypeStruct((N,), F32),
        jax.ShapeDtypeStruct((N,), F32),
    ),
    mesh=_mesh,
    scratch_types=[
        pltpu.VMEM((CW * D,), F32),
        pltpu.VMEM((TPW,), jnp.int32),
        pltpu.VMEM((B * D,), F32),
        pltpu.VMEM((B * D,), F32),
        pltpu.VMEM((B * L,), F32),
        pltpu.VMEM((TPW,), F32),
        pltpu.VMEM((TPW,), F32),
    ],
)(_k1_body)


def _k2_body(psum_hbm, pmax_hbm, pcnt_hbm, wqt_hbm, wk_hbm,
             lnqg_hbm, lnqb_hbm, lnkg_hbm, lnkb_hbm, params_hbm,
             mean_hbm, qpg_hbm, s1c_hbm,
             colbuf, qinbuf, qnbuf, wqbuf, wkbuf, qpgstage, s1cstage,
             pcntbuf, lnqgbuf, lnqbbuf, lnkgbuf, lnkbbuf, paramsbuf):
    w = _wid()

    @pl.when(w < B)
    def _():
        b = w
        pltpu.sync_copy(wqt_hbm, wqbuf)
        pltpu.sync_copy(wk_hbm, wkbuf)
        pltpu.sync_copy(pcnt_hbm, pcntbuf)
        pltpu.sync_copy(lnqg_hbm, lnqgbuf)
        pltpu.sync_copy(lnqb_hbm, lnqbbuf)
        pltpu.sync_copy(lnkg_hbm, lnkgbuf)
        pltpu.sync_copy(lnkb_hbm, lnkbbuf)
        pltpu.sync_copy(params_hbm, paramsbuf)

        for wi in range(NW):
            pltpu.sync_copy(psum_hbm.at[pl.ds(wi * (B * D) + b * D, D)],
                            colbuf.at[pl.ds(wi * D, D)])
        sums = []
        for j in range(NVD):
            a = colbuf[pl.ds(j * L, L)]
            for wi in range(1, NW):
                a = a + colbuf[pl.ds(wi * D + j * L, L)]
            sums.append(a)
        for wi in range(NW):
            pltpu.sync_copy(pmax_hbm.at[pl.ds(wi * (B * D) + b * D, D)],
                            colbuf.at[pl.ds(wi * D, D)])
        maxs = []
        for j in range(NVD):
            a = colbuf[pl.ds(j * L, L)]
            for wi in range(1, NW):
                a = jnp.maximum(a, colbuf[pl.ds(wi * D + j * L, L)])
            maxs.append(a)
        cntv = pcntbuf[pl.ds(b * L, L)]
        for wi in range(1, NW):
            cntv = cntv + pcntbuf[pl.ds(wi * (B * L) + b * L, L)]
        for j in range(NVD):
            qinbuf[pl.ds(j * L, L)] = sums[j] / cntv
            qinbuf[pl.ds(D + j * L, L)] = maxs[j]
        pltpu.sync_copy(qinbuf.at[pl.ds(0, D)], mean_hbm.at[pl.ds(b * D, D)])

        def qstep(gi, accs):
            qrow = qinbuf[pl.ds(gi * L, L)]
            for i in range(L):
                sp = _take(qrow, i)
                accs = tuple(
                    accs[j] + sp * wqbuf[pl.ds((gi * L + i) * D + j * L, L)]
                    for j in range(NVD))
            return accs

        accs = lax.fori_loop(0, 2 * D // L, qstep,
                             tuple(jnp.zeros((L,), F32) for _ in range(NVD)))
        s = accs[0]
        sq = accs[0] * accs[0]
        for j in range(1, NVD):
            s = s + accs[j]
            sq = accs[j] * accs[j] + sq
        musp = _allsum(s) * (1.0 / D)
        var = _allsum(sq) * (1.0 / D) - musp * musp
        rssp = _rsqrt_v(var + 1e-5)
        for j in range(NVD):
            qn = ((accs[j] - musp) * rssp * lnqgbuf[pl.ds(j * L, L)]
                  + lnqbbuf[pl.ds(j * L, L)])
            qnbuf[pl.ds(j * L, L)] = qn

        prow = paramsbuf[pl.ds(0, L)]
        ssp = _take(prow, 2) * np.float32(1.0 / np.sqrt(HD))
        lane = _lane()
        s1c = jnp.zeros((L,), F32)
        for hh in range(H):
            def pstep(g2, accs, hh=hh):
                qnrow = qnbuf[pl.ds(hh * HD + g2 * L, L)]
                for i in range(L):
                    r = hh * HD + g2 * L + i
                    sp = _take(qnrow, i)
                    accs = tuple(
                        accs[j] + sp * wkbuf[pl.ds(r * D + j * L, L)]
                        for j in range(NVD))
                return accs

            paccs = lax.fori_loop(0, HD // L, pstep,
                                  tuple(jnp.zeros((L,), F32)
                                        for _ in range(NVD)))
            qp = [p * ssp for p in paccs]
            s1v = None
            ccv = None
            for j in range(NVD):
                qpg = qp[j] * lnkgbuf[pl.ds(j * L, L)]
                qpgstage[pl.ds(hh * D + j * L, L)] = qpg
                cterm = qp[j] * lnkbbuf[pl.ds(j * L, L)]
                s1v = qpg if s1v is None else s1v + qpg
                ccv = cterm if ccv is None else ccv + cterm
            s1c = jnp.where(lane == hh, _allsum(s1v), s1c)
            s1c = jnp.where(lane == H + hh, _allsum(ccv), s1c)
        s1cstage[pl.ds(0, L)] = s1c
        pltpu.sync_copy(qpgstage, qpg_hbm.at[pl.ds(b * (H * D), H * D)])
        pltpu.sync_copy(s1cstage, s1c_hbm.at[pl.ds(b * L, L)])


_k2 = functools.partial(
    pl.kernel,
    out_type=(
        jax.ShapeDtypeStruct((B * D,), F32),
        jax.ShapeDtypeStruct((B * H * D,), F32),
        jax.ShapeDtypeStruct((B * L,), F32),
    ),
    mesh=_mesh,
    scratch_types=[
        pltpu.VMEM((NW * D,), F32),
        pltpu.VMEM((2 * D,), F32),
        pltpu.VMEM((D,), F32),
        pltpu.VMEM((2 * D * D,), F32),
        pltpu.VMEM((D * D,), F32),
        pltpu.VMEM((H * D,), F32),
        pltpu.VMEM((L,), F32),
        pltpu.VMEM((NW * B * L,), F32),
        pltpu.VMEM((D,), F32),
        pltpu.VMEM((D,), F32),
        pltpu.VMEM((D,), F32),
        pltpu.VMEM((D,), F32),
        pltpu.VMEM((L,), F32),
    ],
)(_k2_body)


def _k3_body(h_hbm, batch_hbm, mu_hbm, rs_hbm, qpg_hbm, s1c_hbm,
             cdr_hbm, iface_hbm, params_hbm,
             log_hbm, pmaxl_hbm,
             hbuf, batchbuf, mubuf, rsbuf, cdbuf, ifbuf, biasbuf,
             qpgbuf, s1cbuf, paramsbuf, logbuf, lmax):
    w = _wid()
    tok0 = w * TPW
    pltpu.sync_copy(batch_hbm.at[pl.ds(tok0, TPW)], batchbuf)
    pltpu.sync_copy(mu_hbm.at[pl.ds(tok0, TPW)], mubuf)
    pltpu.sync_copy(rs_hbm.at[pl.ds(tok0, TPW)], rsbuf)
    pltpu.sync_copy(cdr_hbm.at[pl.ds(tok0, TPW)], cdbuf)
    pltpu.sync_copy(iface_hbm.at[pl.ds(tok0, TPW)], ifbuf)
    pltpu.sync_copy(qpg_hbm, qpgbuf)
    pltpu.sync_copy(s1c_hbm, s1cbuf)
    pltpu.sync_copy(params_hbm, paramsbuf)

    prow = paramsbuf[pl.ds(0, L)]
    cbsp = _take(prow, 0)
    ibsp = _take(prow, 1)
    lane = _lane()

    @pl.loop(0, TPW // L)
    def _(i):
        idx = pl.ds(i * L, L)
        biasbuf[idx] = cbsp * cdbuf[idx] + ibsp * ifbuf[idx]

    @pl.loop(0, B)
    def _(i):
        lmax[pl.ds(i * L, L)] = jnp.full((L,), NEG, F32)

    @pl.loop(0, NCH)
    def _(ch):
        pltpu.sync_copy(h_hbm.at[pl.ds((tok0 + ch * CW) * D, CW * D)], hbuf)
        c0 = ch * CW
        brow0 = batchbuf[pl.ds(c0, L)]
        brow1 = batchbuf[pl.ds(c0 + CW - L, L)]
        uniform = brow0[0] == brow1[L - 1]

        @pl.when(uniform)
        def _():
            seg = brow0[0]
            qb = seg * (H * D)
            qv = [[qpgbuf[pl.ds(qb + hh * D + j * L, L)] for j in range(NVD)]
                  for hh in range(H)]
            s1crow = s1cbuf[pl.ds(seg * L, L)]
            s1l = _gather(s1crow, lane & 3)
            cl = _gather(s1crow, (lane & 3) + H)

            def gloop(g, lm):
                gt = ch * NG + g
                murow = mubuf[pl.ds(gt * L, L)]
                rsrow = rsbuf[pl.ds(gt * L, L)]
                biasrow = biasbuf[pl.ds(gt * L, L)]
                rmrow = rsrow * murow
                for i in range(L):
                    off = (g * L + i) * D
                    hv = [hbuf[pl.ds(off + j * L, L)] for j in range(NVD)]
                    rssp = _take(rsrow, i)
                    lrow = jnp.full((L,), NEG, F32)
                    for hh in range(H):
                        p = hv[0] * qv[hh][0]
                        for j in range(1, NVD):
                            p = hv[j] * qv[hh][j] + p
                        lrow = jnp.where(lane == hh, rssp * _allsum(p), lrow)
                    lrow = (lrow - _take(rmrow, i) * s1l + cl
                            + _take(biasrow, i))
                    logbuf[pl.ds((gt * L + i) * L, L)] = lrow
                    lm = jnp.maximum(lm, lrow)
                return lm

            mi = pl.ds(seg * L, L)
            lmax[mi] = lax.fori_loop(0, NG, gloop, lmax[mi])

        @pl.when(jnp.logical_not(uniform))
        def _():
            @pl.loop(0, NG)
            def _(g):
                gt = ch * NG + g
                batchrow = batchbuf[pl.ds(gt * L, L)]
                murow = mubuf[pl.ds(gt * L, L)]
                rsrow = rsbuf[pl.ds(gt * L, L)]
                biasrow = biasbuf[pl.ds(gt * L, L)]
                rmrow = rsrow * murow
                for i in range(L):
                    off = (g * L + i) * D
                    hv = [hbuf[pl.ds(off + j * L, L)] for j in range(NVD)]
                    seg = batchrow[i]
                    qb = seg * (H * D)
                    rssp = _take(rsrow, i)
                    s1crow = s1cbuf[pl.ds(seg * L, L)]
                    lrow = jnp.full((L,), NEG, F32)
                    for hh in range(H):
                        p = hv[0] * qpgbuf[pl.ds(qb + hh * D, L)]
                        for j in range(1, NVD):
                            p = hv[j] * qpgbuf[pl.ds(qb + hh * D + j * L, L)] + p
                        lrow = jnp.where(lane == hh, rssp * _allsum(p), lrow)
                    s1l = _gather(s1crow, lane & 3)
                    cl = _gather(s1crow, (lane & 3) + H)
                    lrow = (lrow - _take(rmrow, i) * s1l + cl
                            + _take(biasrow, i))
                    logbuf[pl.ds((gt * L + i) * L, L)] = lrow
                    mi = pl.ds(seg * L, L)
                    lmax[mi] = jnp.maximum(lmax[mi], lrow)

    pltpu.sync_copy(logbuf, log_hbm.at[pl.ds(tok0 * L, TPW * L)])
    pltpu.sync_copy(lmax, pmaxl_hbm.at[pl.ds(w * (B * L), B * L)])


_k3 = functools.partial(
    pl.kernel,
    out_type=(
        jax.ShapeDtypeStruct((N * L,), F32),
        jax.ShapeDtypeStruct((NW * B * L,), F32),
    ),
    mesh=_mesh,
    scratch_types=[
        pltpu.VMEM((CW * D,), F32),
        pltpu.VMEM((TPW,), jnp.int32),
        pltpu.VMEM((TPW,), F32),
        pltpu.VMEM((TPW,), F32),
        pltpu.VMEM((TPW,), F32),
        pltpu.VMEM((TPW,), F32),
        pltpu.VMEM((TPW,), F32),
        pltpu.VMEM((B * H * D,), F32),
        pltpu.VMEM((B * L,), F32),
        pltpu.VMEM((L,), F32),
        pltpu.VMEM((TPW * L,), F32),
        pltpu.VMEM((B * L,), F32),
    ],
)(_k3_body)


def _k4_body(h_hbm, batch_hbm, mu_hbm, rs_hbm, log_hbm, pmaxl_hbm,
             pp_hbm, sp_hbm, t1_hbm,
             hbuf, batchbuf, mubuf, rsbuf, logbuf, pmaxlbuf, mbuf,
             pacc, sacc, t1acc):
    w = _wid()
    tok0 = w * TPW
    pltpu.sync_copy(batch_hbm.at[pl.ds(tok0, TPW)], batchbuf)
    pltpu.sync_copy(mu_hbm.at[pl.ds(tok0, TPW)], mubuf)
    pltpu.sync_copy(rs_hbm.at[pl.ds(tok0, TPW)], rsbuf)
    pltpu.sync_copy(pmaxl_hbm, pmaxlbuf)

    @pl.loop(0, B)
    def _(i):
        a = pmaxlbuf[pl.ds(i * L, L)]
        for wi in range(1, NW):
            a = jnp.maximum(a, pmaxlbuf[pl.ds(wi * (B * L) + i * L, L)])
        mbuf[pl.ds(i * L, L)] = a

    @pl.loop(0, B * H * D // L)
    def _(i):
        pacc[pl.ds(i * L, L)] = jnp.zeros((L,), F32)

    @pl.loop(0, B)
    def _(i):
        sacc[pl.ds(i * L, L)] = jnp.zeros((L,), F32)
        t1acc[pl.ds(i * L, L)] = jnp.zeros((L,), F32)

    @pl.loop(0, NCH)
    def _(ch):
        pltpu.sync_copy(h_hbm.at[pl.ds((tok0 + ch * CW) * D, CW * D)], hbuf)
        pltpu.sync_copy(log_hbm.at[pl.ds((tok0 + ch * CW) * L, CW * L)],
                        logbuf)
        c0 = ch * CW
        brow0 = batchbuf[pl.ds(c0, L)]
        brow1 = batchbuf[pl.ds(c0 + CW - L, L)]
        uniform = brow0[0] == brow1[L - 1]

        @pl.when(uniform)
        def _():
            seg = brow0[0]
            pb0 = seg * (H * D)
            si = pl.ds(seg * L, L)
            mrow = mbuf[si]

            def gloop(g, carry):
                ps = [list(carry[hh * NVD:(hh + 1) * NVD]) for hh in range(H)]
                srow = carry[H * NVD]
                t1row = carry[H * NVD + 1]
                gt = ch * NG + g
                murow = mubuf[pl.ds(gt * L, L)]
                rsrow = rsbuf[pl.ds(gt * L, L)]
                rmrow = rsrow * murow
                for i in range(L):
                    off = (g * L + i) * D
                    hv = [hbuf[pl.ds(off + j * L, L)] for j in range(NVD)]
                    lrow = logbuf[pl.ds((g * L + i) * L, L)]
                    erow = jnp.exp(lrow - mrow)
                    srow = srow + erow
                    t1row = erow * _take(rmrow, i) + t1row
                    rssp = _take(rsrow, i)
                    for hh in range(H):
                        wr = _take(erow, hh) * rssp
                        for j in range(NVD):
                            ps[hh][j] = wr * hv[j] + ps[hh][j]
                return (*ps[0], *ps[1], *ps[2], *ps[3], srow, t1row)

            init = (tuple(pacc[pl.ds(pb0 + hh * D + j * L, L)]
                          for hh in range(H) for j in range(NVD))
                    + (sacc[si], t1acc[si]))
            res = lax.fori_loop(0, NG, gloop, init)
            for hh in range(H):
                for j in range(NVD):
                    pacc[pl.ds(pb0 + hh * D + j * L, L)] = res[hh * NVD + j]
            sacc[si] = res[H * NVD]
            t1acc[si] = res[H * NVD + 1]

        @pl.when(jnp.logical_not(uniform))
        def _():
            @pl.loop(0, NG)
            def _(g):
                gt = ch * NG + g
                batchrow = batchbuf[pl.ds(gt * L, L)]
                murow = mubuf[pl.ds(gt * L, L)]
                rsrow = rsbuf[pl.ds(gt * L, L)]
                rmrow = rsrow * murow
                for i in range(L):
                    off = (g * L + i) * D
                    hv = [hbuf[pl.ds(off + j * L, L)] for j in range(NVD)]
                    seg = batchrow[i]
                    rmsp = _take(rmrow, i)
                    rssp = _take(rsrow, i)
                    lrow = logbuf[pl.ds((g * L + i) * L, L)]
                    mrow = mbuf[pl.ds(seg * L, L)]
                    erow = jnp.exp(lrow - mrow)
                    si = pl.ds(seg * L, L)
                    sacc[si] = sacc[si] + erow
                    t1acc[si] = erow * rmsp + t1acc[si]
                    for hh in range(H):
                        wr = _take(erow, hh) * rssp
                        pb = seg * (H * D) + hh * D
                        for j in range(NVD):
                            idx = pl.ds(pb + j * L, L)
                            pacc[idx] = wr * hv[j] + pacc[idx]

    pltpu.sync_copy(pacc, pp_hbm.at[pl.ds(w * (B * H * D), B * H * D)])
    pltpu.sync_copy(sacc, sp_hbm.at[pl.ds(w * (B * L), B * L)])
    pltpu.sync_copy(t1acc, t1_hbm.at[pl.ds(w * (B * L), B * L)])


_k4 = functools.partial(
    pl.kernel,
    out_type=(
        jax.ShapeDtypeStruct((NW * B * H * D,), F32),
        jax.ShapeDtypeStruct((NW * B * L,), F32),
        jax.ShapeDtypeStruct((NW * B * L,), F32),
    ),
    mesh=_mesh,
    scratch_types=[
        pltpu.VMEM((CW * D,), F32),
        pltpu.VMEM((TPW,), jnp.int32),
        pltpu.VMEM((TPW,), F32),
        pltpu.VMEM((TPW,), F32),
        pltpu.VMEM((CW * L,), F32),
        pltpu.VMEM((NW * B * L,), F32),
        pltpu.VMEM((B * L,), F32),
        pltpu.VMEM((B * H * D,), F32),
        pltpu.VMEM((B * L,), F32),
        pltpu.VMEM((B * L,), F32),
    ],
)(_k4_body)


def _k5_body(pp_hbm, sp_hbm, t1_hbm, mean_hbm, wv_hbm, woutt_hbm, wrest_hbm,
             lnkg_hbm, lnkb_hbm,
             out_hbm,
             pbuf, sbuf, t1buf, meanbuf, wvbuf, woutbuf, wresbuf,
             gkbuf, bkbuf, abuf, obuf):
    w = _wid()
    b = w // 2
    o0 = (w % 2) * (D // 2)
    for wi in range(NW):
        pltpu.sync_copy(pp_hbm.at[pl.ds(wi * (B * H * D) + b * (H * D), H * D)],
                        pbuf.at[pl.ds(wi * (H * D), H * D)])
    pltpu.sync_copy(sp_hbm, sbuf)
    pltpu.sync_copy(t1_hbm, t1buf)
    pltpu.sync_copy(mean_hbm.at[pl.ds(b * D, D)], meanbuf)
    pltpu.sync_copy(wv_hbm, wvbuf)
    pltpu.sync_copy(woutt_hbm, woutbuf)
    pltpu.sync_copy(wrest_hbm, wresbuf)
    pltpu.sync_copy(lnkg_hbm, gkbuf)
    pltpu.sync_copy(lnkb_hbm, bkbuf)

    srow = sbuf[pl.ds(b * L, L)]
    t1row = t1buf[pl.ds(b * L, L)]
    for wi in range(1, NW):
        srow = srow + sbuf[pl.ds(wi * (B * L) + b * L, L)]
        t1row = t1row + t1buf[pl.ds(wi * (B * L) + b * L, L)]

    for hh in range(H):
        sv = _take(srow, hh)
        t1v = _take(t1row, hh)
        for j in range(NVD):
            pm = pbuf[pl.ds(hh * D + j * L, L)]
            for wi in range(1, NW):
                pm = pm + pbuf[pl.ds(wi * (H * D) + hh * D + j * L, L)]
            gk = gkbuf[pl.ds(j * L, L)]
            bk = bkbuf[pl.ds(j * L, L)]
            abuf[pl.ds(hh * D + j * L, L)] = (gk * (pm - t1v) + bk * sv) / sv

    zero4 = tuple(jnp.zeros((L,), F32) for _ in range(4))
    oacc = zero4
    for hh in range(H):
        av = [abuf[pl.ds(hh * D + j * L, L)] for j in range(NVD)]

        def kstep(k2, oaccs, hh=hh, av=av):
            k = hh * HD + k2
            p = av[0] * wvbuf[pl.ds(k * D, L)]
            for j in range(1, NVD):
                p = av[j] * wvbuf[pl.ds(k * D + j * L, L)] + p
            gs = _allsum(p)
            return tuple(oaccs[m] + gs * woutbuf[pl.ds(k * D + o0 + m * L, L)]
                         for m in range(4))

        oacc = lax.fori_loop(0, HD, kstep, oacc)

    def rstep(gk2, raccs):
        mrow = meanbuf[pl.ds(gk2 * L, L)]
        for i in range(L):
            ms = _take(mrow, i)
            k = gk2 * L + i
            raccs = tuple(
                raccs[m] + ms * wresbuf[pl.ds(k * D + o0 + m * L, L)]
                for m in range(4))
        return raccs

    racc = lax.fori_loop(0, D // L, rstep, zero4)
    for m in range(4):
        obuf[pl.ds(m * L, L)] = oacc[m] + np.float32(MRS) * racc[m]
    pltpu.sync_copy(obuf, out_hbm.at[pl.ds(b * D + o0, D // 2)])


_k5 = functools.partial(
    pl.kernel,
    out_type=jax.ShapeDtypeStruct((B * D,), F32),
    mesh=_mesh,
    scratch_types=[
        pltpu.VMEM((NW * H * D,), F32),
        pltpu.VMEM((NW * B * L,), F32),
        pltpu.VMEM((NW * B * L,), F32),
        pltpu.VMEM((D,), F32),
        pltpu.VMEM((D * D,), F32),
        pltpu.VMEM((D * D,), F32),
        pltpu.VMEM((D * D,), F32),
        pltpu.VMEM((D,), F32),
        pltpu.VMEM((D,), F32),
        pltpu.VMEM((H * D,), F32),
        pltpu.VMEM((D // 2,), F32),
    ],
)(_k5_body)


def kernel(h, batch, cdr_mask, iface_mask, Wk, Wv, Wq, Wres, Wout,
           ln_kv_g, ln_kv_b, ln_q_g, ln_q_b, cdr_bias, iface_bias,
           logit_scale):
    h_flat = h.reshape(-1)
    cdrf = cdr_mask.astype(F32)
    ifacef = iface_mask.astype(F32)
    params = jnp.concatenate([
        jnp.stack([cdr_bias.astype(F32), iface_bias.astype(F32),
                   logit_scale.astype(F32)]),
        jnp.zeros((L - 3,), F32),
    ])
    wqt = Wq.T.reshape(-1)
    wk = Wk.reshape(-1)
    wv = Wv.reshape(-1)
    woutt = Wout.T.reshape(-1)
    wrest = Wres.T.reshape(-1)

    psum, pmax, pcnt, mu, rs = _k1(h_flat, batch)
    mean, qpg, s1c = _k2(psum, pmax, pcnt, wqt, wk,
                         ln_q_g, ln_q_b, ln_kv_g, ln_kv_b, params)
    logits, pmaxl = _k3(h_flat, batch, mu, rs, qpg, s1c, cdrf, ifacef, params)
    pp, sp, t1 = _k4(h_flat, batch, mu, rs, logits, pmaxl)
    out = _k5(pp, sp, t1, mean, wv, woutt, wrest, ln_kv_g, ln_kv_b)
    return out.reshape(B, D)


# speedup vs baseline: 19.6053x; 1.4420x over previous
import functools

import jax
import jax.numpy as jnp
import numpy as np
from jax import lax
from jax.experimental import pallas as pl
from jax.experimental.pallas import tpu as pltpu
from jax.experimental.pallas import tpu_sc as plsc

N = 32768
D = 128
H = 4
HD = D // H
B = 16
MRS = 0.2

NC = 2
NS = 16
NW = NC * NS
L = 16
NVD = D // L
TPW = N // NW
CW = 128
NCH = TPW // CW
NG = CW // L
F32 = jnp.float32
NEG = -1e30

_mesh = plsc.VectorSubcoreMesh(
    core_axis_name="c", subcore_axis_name="s", num_cores=NC, num_subcores=NS)


def _wid():
    return lax.axis_index("s") * NC + lax.axis_index("c")


def _splat(x):
    return jnp.full((L,), x, F32)


def _lane():
    return lax.broadcasted_iota(jnp.int32, (L,), 0)


_GDN = lax.GatherDimensionNumbers(
    offset_dims=(), collapsed_slice_dims=(0,), start_index_map=(0,))


def _gather(v, idx):
    return lax.gather(v, idx[:, None], _GDN, slice_sizes=(1,),
                      mode=lax.GatherScatterMode.PROMISE_IN_BOUNDS)


def _take(v, i):
    return _gather(v, jnp.full((L,), i, jnp.int32))


def _allsum(v):
    lane = _lane()
    for k in (8, 4, 2, 1):
        v = v + _gather(v, lane ^ k)
    return v


def _rsqrt_v(x):
    i = lax.bitcast_convert_type(x, jnp.int32)
    i = jnp.full((L,), 0x5F3759DF, jnp.int32) - (i >> 1)
    y = lax.bitcast_convert_type(i, F32)
    for _ in range(3):
        y = y * (1.5 - 0.5 * x * y * y)
    return y


def _k1_body(h_hbm, batch_hbm,
             psum_hbm, pmax_hbm, pcnt_hbm, mu_hbm, rs_hbm,
             hbufA, hbufB, batchbuf, accsum, accmax, acccnt, mubuf, rsbuf,
             semA, semB, semO):
    w = _wid()
    tok0 = w * TPW
    pltpu.sync_copy(batch_hbm.at[pl.ds(tok0, TPW)], batchbuf)
    lane = _lane()

    @pl.loop(0, B * NVD)
    def _(i):
        accsum[pl.ds(i * L, L)] = jnp.zeros((L,), F32)
        accmax[pl.ds(i * L, L)] = jnp.full((L,), NEG, F32)

    @pl.loop(0, B)
    def _(i):
        acccnt[pl.ds(i * L, L)] = jnp.zeros((L,), F32)

    def _chunk(ch, hbuf):
        c0 = ch * CW
        brow0 = batchbuf[pl.ds(c0, L)]
        brow1 = batchbuf[pl.ds(c0 + CW - L, L)]
        uniform = brow0[0] == brow1[L - 1]

        @pl.when(uniform)
        def _():
            seg = brow0[0]
            sbase = seg * D

            def gloop(g, carry):
                accs = list(carry[:NVD])
                accm = list(carry[NVD:2 * NVD])
                cnt = carry[2 * NVD]
                gt = ch * NG + g
                murow = jnp.zeros((L,), F32)
                rsrow = jnp.zeros((L,), F32)
                for i in range(L):
                    off = (g * L + i) * D
                    hv = [hbuf[pl.ds(off + j * L, L)] for j in range(NVD)]
                    s = hv[0]
                    sq = hv[0] * hv[0]
                    for j in range(1, NVD):
                        s = s + hv[j]
                        sq = hv[j] * hv[j] + sq
                    mu = _allsum(s) * (1.0 / D)
                    var = _allsum(sq) * (1.0 / D) - mu * mu
                    rs = _rsqrt_v(var + 1e-5)
                    murow = jnp.where(lane == i, mu, murow)
                    rsrow = jnp.where(lane == i, rs, rsrow)
                    for j in range(NVD):
                        accs[j] = accs[j] + hv[j]
                        accm[j] = jnp.maximum(accm[j], hv[j])
                mubuf[pl.ds(gt * L, L)] = murow
                rsbuf[pl.ds(gt * L, L)] = rsrow
                return (*accs, *accm, cnt + F32(L))

            init = (tuple(accsum[pl.ds(sbase + j * L, L)] for j in range(NVD))
                    + tuple(accmax[pl.ds(sbase + j * L, L)]
                            for j in range(NVD))
                    + (acccnt[pl.ds(seg * L, L)],))
            res = lax.fori_loop(0, NG, gloop, init)
            for j in range(NVD):
                accsum[pl.ds(sbase + j * L, L)] = res[j]
                accmax[pl.ds(sbase + j * L, L)] = res[NVD + j]
            acccnt[pl.ds(seg * L, L)] = res[2 * NVD]

        @pl.when(jnp.logical_not(uniform))
        def _():
            @pl.loop(0, NG)
            def _(g):
                gt = ch * NG + g
                batchrow = batchbuf[pl.ds(gt * L, L)]
                murow = jnp.zeros((L,), F32)
                rsrow = jnp.zeros((L,), F32)
                for i in range(L):
                    off = (g * L + i) * D
                    hv = [hbuf[pl.ds(off + j * L, L)] for j in range(NVD)]
                    s = hv[0]
                    sq = hv[0] * hv[0]
                    for j in range(1, NVD):
                        s = s + hv[j]
                        sq = hv[j] * hv[j] + sq
                    mu = _allsum(s) * (1.0 / D)
                    var = _allsum(sq) * (1.0 / D) - mu * mu
                    rs = _rsqrt_v(var + 1e-5)
                    murow = jnp.where(lane == i, mu, murow)
                    rsrow = jnp.where(lane == i, rs, rsrow)
                    seg = batchrow[i]
                    sbase = seg * D
                    for j in range(NVD):
                        idx = pl.ds(sbase + j * L, L)
                        accsum[idx] = accsum[idx] + hv[j]
                        accmax[idx] = jnp.maximum(accmax[idx], hv[j])
                    ci = pl.ds(seg * L, L)
                    acccnt[ci] = acccnt[ci] + 1.0
                mubuf[pl.ds(gt * L, L)] = murow
                rsbuf[pl.ds(gt * L, L)] = rsrow

    pltpu.async_copy(h_hbm.at[pl.ds(tok0 * D, CW * D)], hbufA, semA)

    @pl.loop(0, NCH, step=2)
    def _(ch):
        pltpu.async_copy(
            h_hbm.at[pl.ds((tok0 + (ch + 1) * CW) * D, CW * D)], hbufB, semB)
        pltpu.make_async_copy(h_hbm.at[pl.ds(0, CW * D)], hbufA, semA).wait()
        _chunk(ch, hbufA)

        @pl.when(ch + 2 < NCH)
        def _():
            pltpu.async_copy(
                h_hbm.at[pl.ds((tok0 + (ch + 2) * CW) * D, CW * D)],
                hbufA, semA)

        pltpu.make_async_copy(h_hbm.at[pl.ds(0, CW * D)], hbufB, semB).wait()
        _chunk(ch + 1, hbufB)

    cps = []
    for b in range(B):
        cps.append(pltpu.async_copy(
            accsum.at[pl.ds(b * D, D)],
            psum_hbm.at[pl.ds((b * NW + w) * D, D)], semO))
        cps.append(pltpu.async_copy(
            accmax.at[pl.ds(b * D, D)],
            pmax_hbm.at[pl.ds((b * NW + w) * D, D)], semO))
    cps.append(pltpu.async_copy(
        acccnt, pcnt_hbm.at[pl.ds(w * (B * L), B * L)], semO))
    cps.append(pltpu.async_copy(mubuf, mu_hbm.at[pl.ds(tok0, TPW)], semO))
    cps.append(pltpu.async_copy(rsbuf, rs_hbm.at[pl.ds(tok0, TPW)], semO))
    for cp in cps:
        cp.wait()


_k1 = functools.partial(
    pl.kernel,
    out_type=(
        jax.ShapeDtypeStruct((NW * B * D,), F32),
        jax.ShapeDtypeStruct((NW * B * D,), F32),
        jax.ShapeDtypeStruct((NW * B * L,), F32),
        jax.ShapeDtypeStruct((N,), F32),
        jax.ShapeDtypeStruct((N,), F32),
    ),
    mesh=_mesh,
    scratch_types=[
        pltpu.VMEM((CW * D,), F32),
        pltpu.VMEM((CW * D,), F32),
        pltpu.VMEM((TPW,), jnp.int32),
        pltpu.VMEM((B * D,), F32),
        pltpu.VMEM((B * D,), F32),
        pltpu.VMEM((B * L,), F32),
        pltpu.VMEM((TPW,), F32),
        pltpu.VMEM((TPW,), F32),
        pltpu.SemaphoreType.DMA,
        pltpu.SemaphoreType.DMA,
        pltpu.SemaphoreType.DMA,
    ],
)(_k1_body)


def _k2_body(psum_hbm, pmax_hbm, pcnt_hbm, wqt_hbm, wk_hbm,
             lnqg_hbm, lnqb_hbm, lnkg_hbm, lnkb_hbm, params_hbm,
             mean_hbm, qpg_hbm, s1c_hbm,
             colbuf, colbuf2, qinbuf, qnbuf, wqbuf, wkbuf, qpgstage,
             s1cstage, pcntbuf, lnqgbuf, lnqbbuf, lnkgbuf, lnkbbuf,
             paramsbuf, sem):
    w = _wid()

    @pl.when(w < B)
    def _():
        b = w
        cps = [
            pltpu.async_copy(wqt_hbm, wqbuf, sem),
            pltpu.async_copy(wk_hbm, wkbuf, sem),
            pltpu.async_copy(pcnt_hbm, pcntbuf, sem),
            pltpu.async_copy(lnqg_hbm, lnqgbuf, sem),
            pltpu.async_copy(lnqb_hbm, lnqbbuf, sem),
            pltpu.async_copy(lnkg_hbm, lnkgbuf, sem),
            pltpu.async_copy(lnkb_hbm, lnkbbuf, sem),
            pltpu.async_copy(params_hbm, paramsbuf, sem),
            pltpu.async_copy(psum_hbm.at[pl.ds(b * (NW * D), NW * D)],
                             colbuf, sem),
            pltpu.async_copy(pmax_hbm.at[pl.ds(b * (NW * D), NW * D)],
                             colbuf2, sem),
        ]
        for cp in cps:
            cp.wait()

        sums = []
        for j in range(NVD):
            a = colbuf[pl.ds(j * L, L)]
            for wi in range(1, NW):
                a = a + colbuf[pl.ds(wi * D + j * L, L)]
            sums.append(a)
        maxs = []
        for j in range(NVD):
            a = colbuf2[pl.ds(j * L, L)]
            for wi in range(1, NW):
                a = jnp.maximum(a, colbuf2[pl.ds(wi * D + j * L, L)])
            maxs.append(a)
        cntv = pcntbuf[pl.ds(b * L, L)]
        for wi in range(1, NW):
            cntv = cntv + pcntbuf[pl.ds(wi * (B * L) + b * L, L)]
        for j in range(NVD):
            qinbuf[pl.ds(j * L, L)] = sums[j] / cntv
            qinbuf[pl.ds(D + j * L, L)] = maxs[j]
        pltpu.sync_copy(qinbuf.at[pl.ds(0, D)], mean_hbm.at[pl.ds(b * D, D)])

        def qstep(gi, accs):
            qrow = qinbuf[pl.ds(gi * L, L)]
            for i in range(L):
                sp = _take(qrow, i)
                accs = tuple(
                    accs[j] + sp * wqbuf[pl.ds((gi * L + i) * D + j * L, L)]
                    for j in range(NVD))
            return accs

        accs = lax.fori_loop(0, 2 * D // L, qstep,
                             tuple(jnp.zeros((L,), F32) for _ in range(NVD)))
        s = accs[0]
        sq = accs[0] * accs[0]
        for j in range(1, NVD):
            s = s + accs[j]
            sq = accs[j] * accs[j] + sq
        musp = _allsum(s) * (1.0 / D)
        var = _allsum(sq) * (1.0 / D) - musp * musp
        rssp = _rsqrt_v(var + 1e-5)
        for j in range(NVD):
            qn = ((accs[j] - musp) * rssp * lnqgbuf[pl.ds(j * L, L)]
                  + lnqbbuf[pl.ds(j * L, L)])
            qnbuf[pl.ds(j * L, L)] = qn

        prow = paramsbuf[pl.ds(0, L)]
        ssp = _take(prow, 2) * np.float32(1.0 / np.sqrt(HD))
        lane = _lane()
        s1c = jnp.zeros((L,), F32)
        for hh in range(H):
            def pstep(g2, accs, hh=hh):
                qnrow = qnbuf[pl.ds(hh * HD + g2 * L, L)]
                for i in range(L):
                    r = hh * HD + g2 * L + i
                    sp = _take(qnrow, i)
                    accs = tuple(
                        accs[j] + sp * wkbuf[pl.ds(r * D + j * L, L)]
                        for j in range(NVD))
                return accs

            paccs = lax.fori_loop(0, HD // L, pstep,
                                  tuple(jnp.zeros((L,), F32)
                                        for _ in range(NVD)))
            qp = [p * ssp for p in paccs]
            s1v = None
            ccv = None
            for j in range(NVD):
                qpg = qp[j] * lnkgbuf[pl.ds(j * L, L)]
                qpgstage[pl.ds(hh * D + j * L, L)] = qpg
                cterm = qp[j] * lnkbbuf[pl.ds(j * L, L)]
                s1v = qpg if s1v is None else s1v + qpg
                ccv = cterm if ccv is None else ccv + cterm
            s1c = jnp.where(lane == hh, _allsum(s1v), s1c)
            s1c = jnp.where(lane == H + hh, _allsum(ccv), s1c)
        s1cstage[pl.ds(0, L)] = s1c
        pltpu.sync_copy(qpgstage, qpg_hbm.at[pl.ds(b * (H * D), H * D)])
        pltpu.sync_copy(s1cstage, s1c_hbm.at[pl.ds(b * L, L)])


_k2 = functools.partial(
    pl.kernel,
    out_type=(
        jax.ShapeDtypeStruct((B * D,), F32),
        jax.ShapeDtypeStruct((B * H * D,), F32),
        jax.ShapeDtypeStruct((B * L,), F32),
    ),
    mesh=_mesh,
    scratch_types=[
        pltpu.VMEM((NW * D,), F32),
        pltpu.VMEM((NW * D,), F32),
        pltpu.VMEM((2 * D,), F32),
        pltpu.VMEM((D,), F32),
        pltpu.VMEM((2 * D * D,), F32),
        pltpu.VMEM((D * D,), F32),
        pltpu.VMEM((H * D,), F32),
        pltpu.VMEM((L,), F32),
        pltpu.VMEM((NW * B * L,), F32),
        pltpu.VMEM((D,), F32),
        pltpu.VMEM((D,), F32),
        pltpu.VMEM((D,), F32),
        pltpu.VMEM((D,), F32),
        pltpu.VMEM((L,), F32),
        pltpu.SemaphoreType.DMA,
    ],
)(_k2_body)


def _k3_body(h_hbm, batch_hbm, mu_hbm, rs_hbm, qpg_hbm, s1c_hbm,
             cdr_hbm, iface_hbm, params_hbm,
             log_hbm, pmaxl_hbm,
             hbufA, hbufB, batchbuf, mubuf, rsbuf, cdbuf, ifbuf, biasbuf,
             qpgbuf, s1cbuf, paramsbuf, logbuf, lmax, semA, semB, sem):
    w = _wid()
    tok0 = w * TPW
    pltpu.async_copy(h_hbm.at[pl.ds(tok0 * D, CW * D)], hbufA, semA)
    cps = [
        pltpu.async_copy(batch_hbm.at[pl.ds(tok0, TPW)], batchbuf, sem),
        pltpu.async_copy(mu_hbm.at[pl.ds(tok0, TPW)], mubuf, sem),
        pltpu.async_copy(rs_hbm.at[pl.ds(tok0, TPW)], rsbuf, sem),
        pltpu.async_copy(cdr_hbm.at[pl.ds(tok0, TPW)], cdbuf, sem),
        pltpu.async_copy(iface_hbm.at[pl.ds(tok0, TPW)], ifbuf, sem),
        pltpu.async_copy(qpg_hbm, qpgbuf, sem),
        pltpu.async_copy(s1c_hbm, s1cbuf, sem),
        pltpu.async_copy(params_hbm, paramsbuf, sem),
    ]
    for cp in cps:
        cp.wait()

    prow = paramsbuf[pl.ds(0, L)]
    cbsp = _take(prow, 0)
    ibsp = _take(prow, 1)
    lane = _lane()

    @pl.loop(0, TPW // L)
    def _(i):
        idx = pl.ds(i * L, L)
        biasbuf[idx] = cbsp * cdbuf[idx] + ibsp * ifbuf[idx]

    @pl.loop(0, B)
    def _(i):
        lmax[pl.ds(i * L, L)] = jnp.full((L,), NEG, F32)

    def _chunk(ch, hbuf):
        c0 = ch * CW
        brow0 = batchbuf[pl.ds(c0, L)]
        brow1 = batchbuf[pl.ds(c0 + CW - L, L)]
        uniform = brow0[0] == brow1[L - 1]

        @pl.when(uniform)
        def _():
            seg = brow0[0]
            qb = seg * (H * D)
            qv = [[qpgbuf[pl.ds(qb + hh * D + j * L, L)] for j in range(NVD)]
                  for hh in range(H)]
            s1crow = s1cbuf[pl.ds(seg * L, L)]
            s1l = _gather(s1crow, lane & 3)
            cl = _gather(s1crow, (lane & 3) + H)

            def gloop(g, lm):
                gt = ch * NG + g
                murow = mubuf[pl.ds(gt * L, L)]
                rsrow = rsbuf[pl.ds(gt * L, L)]
                biasrow = biasbuf[pl.ds(gt * L, L)]
                rmrow = rsrow * murow
                for i in range(L):
                    off = (g * L + i) * D
                    hv = [hbuf[pl.ds(off + j * L, L)] for j in range(NVD)]
                    rssp = _take(rsrow, i)
                    lrow = jnp.full((L,), NEG, F32)
                    for hh in range(H):
                        p = hv[0] * qv[hh][0]
                        for j in range(1, NVD):
                            p = hv[j] * qv[hh][j] + p
                        lrow = jnp.where(lane == hh, rssp * _allsum(p), lrow)
                    lrow = (lrow - _take(rmrow, i) * s1l + cl
                            + _take(biasrow, i))
                    logbuf[pl.ds((gt * L + i) * L, L)] = lrow
                    lm = jnp.maximum(lm, lrow)
                return lm

            mi = pl.ds(seg * L, L)
            lmax[mi] = lax.fori_loop(0, NG, gloop, lmax[mi])

        @pl.when(jnp.logical_not(uniform))
        def _():
            @pl.loop(0, NG)
            def _(g):
                gt = ch * NG + g
                batchrow = batchbuf[pl.ds(gt * L, L)]
                murow = mubuf[pl.ds(gt * L, L)]
                rsrow = rsbuf[pl.ds(gt * L, L)]
                biasrow = biasbuf[pl.ds(gt * L, L)]
                rmrow = rsrow * murow
                for i in range(L):
                    off = (g * L + i) * D
                    hv = [hbuf[pl.ds(off + j * L, L)] for j in range(NVD)]
                    seg = batchrow[i]
                    qb = seg * (H * D)
                    rssp = _take(rsrow, i)
                    s1crow = s1cbuf[pl.ds(seg * L, L)]
                    lrow = jnp.full((L,), NEG, F32)
                    for hh in range(H):
                        p = hv[0] * qpgbuf[pl.ds(qb + hh * D, L)]
                        for j in range(1, NVD):
                            p = hv[j] * qpgbuf[pl.ds(qb + hh * D + j * L, L)] + p
                        lrow = jnp.where(lane == hh, rssp * _allsum(p), lrow)
                    s1l = _gather(s1crow, lane & 3)
                    cl = _gather(s1crow, (lane & 3) + H)
                    lrow = (lrow - _take(rmrow, i) * s1l + cl
                            + _take(biasrow, i))
                    logbuf[pl.ds((gt * L + i) * L, L)] = lrow
                    mi = pl.ds(seg * L, L)
                    lmax[mi] = jnp.maximum(lmax[mi], lrow)

    @pl.loop(0, NCH, step=2)
    def _(ch):
        pltpu.async_copy(
            h_hbm.at[pl.ds((tok0 + (ch + 1) * CW) * D, CW * D)], hbufB, semB)
        pltpu.make_async_copy(h_hbm.at[pl.ds(0, CW * D)], hbufA, semA).wait()
        _chunk(ch, hbufA)

        @pl.when(ch + 2 < NCH)
        def _():
            pltpu.async_copy(
                h_hbm.at[pl.ds((tok0 + (ch + 2) * CW) * D, CW * D)],
                hbufA, semA)

        pltpu.make_async_copy(h_hbm.at[pl.ds(0, CW * D)], hbufB, semB).wait()
        _chunk(ch + 1, hbufB)

    cpo = [
        pltpu.async_copy(logbuf, log_hbm.at[pl.ds(tok0 * L, TPW * L)], sem),
        pltpu.async_copy(lmax, pmaxl_hbm.at[pl.ds(w * (B * L), B * L)], sem),
    ]
    for cp in cpo:
        cp.wait()


_k3 = functools.partial(
    pl.kernel,
    out_type=(
        jax.ShapeDtypeStruct((N * L,), F32),
        jax.ShapeDtypeStruct((NW * B * L,), F32),
    ),
    mesh=_mesh,
    scratch_types=[
        pltpu.VMEM((CW * D,), F32),
        pltpu.VMEM((CW * D,), F32),
        pltpu.VMEM((TPW,), jnp.int32),
        pltpu.VMEM((TPW,), F32),
        pltpu.VMEM((TPW,), F32),
        pltpu.VMEM((TPW,), F32),
        pltpu.VMEM((TPW,), F32),
        pltpu.VMEM((TPW,), F32),
        pltpu.VMEM((B * H * D,), F32),
        pltpu.VMEM((B * L,), F32),
        pltpu.VMEM((L,), F32),
        pltpu.VMEM((TPW * L,), F32),
        pltpu.VMEM((B * L,), F32),
        pltpu.SemaphoreType.DMA,
        pltpu.SemaphoreType.DMA,
        pltpu.SemaphoreType.DMA,
    ],
)(_k3_body)


def _k4_body(h_hbm, batch_hbm, mu_hbm, rs_hbm, log_hbm, pmaxl_hbm,
             pp_hbm, sp_hbm, t1_hbm,
             hbufA, hbufB, batchbuf, mubuf, rsbuf, logbufA, logbufB,
             pmaxlbuf, mbuf, pacc, sacc, t1acc, semA, semB, sem):
    w = _wid()
    tok0 = w * TPW
    pltpu.async_copy(h_hbm.at[pl.ds(tok0 * D, CW * D)], hbufA, semA)
    pltpu.async_copy(log_hbm.at[pl.ds(tok0 * L, CW * L)], logbufA, semA)
    cps = [
        pltpu.async_copy(batch_hbm.at[pl.ds(tok0, TPW)], batchbuf, sem),
        pltpu.async_copy(mu_hbm.at[pl.ds(tok0, TPW)], mubuf, sem),
        pltpu.async_copy(rs_hbm.at[pl.ds(tok0, TPW)], rsbuf, sem),
        pltpu.async_copy(pmaxl_hbm, pmaxlbuf, sem),
    ]
    for cp in cps:
        cp.wait()

    @pl.loop(0, B)
    def _(i):
        a = pmaxlbuf[pl.ds(i * L, L)]
        for wi in range(1, NW):
            a = jnp.maximum(a, pmaxlbuf[pl.ds(wi * (B * L) + i * L, L)])
        mbuf[pl.ds(i * L, L)] = a

    @pl.loop(0, B * H * D // L)
    def _(i):
        pacc[pl.ds(i * L, L)] = jnp.zeros((L,), F32)

    @pl.loop(0, B)
    def _(i):
        sacc[pl.ds(i * L, L)] = jnp.zeros((L,), F32)
        t1acc[pl.ds(i * L, L)] = jnp.zeros((L,), F32)

    def _chunk(ch, hbuf, logbuf):
        c0 = ch * CW
        brow0 = batchbuf[pl.ds(c0, L)]
        brow1 = batchbuf[pl.ds(c0 + CW - L, L)]
        uniform = brow0[0] == brow1[L - 1]

        @pl.when(uniform)
        def _():
            seg = brow0[0]
            pb0 = seg * (H * D)
            si = pl.ds(seg * L, L)
            mrow = mbuf[si]

            def gloop(g, carry):
                ps = [list(carry[hh * NVD:(hh + 1) * NVD]) for hh in range(H)]
                srow = carry[H * NVD]
                t1row = carry[H * NVD + 1]
                gt = ch * NG + g
                murow = mubuf[pl.ds(gt * L, L)]
                rsrow = rsbuf[pl.ds(gt * L, L)]
                rmrow = rsrow * murow
                for i in range(L):
                    off = (g * L + i) * D
                    hv = [hbuf[pl.ds(off + j * L, L)] for j in range(NVD)]
                    lrow = logbuf[pl.ds((g * L + i) * L, L)]
                    erow = jnp.exp(lrow - mrow)
                    srow = srow + erow
                    t1row = erow * _take(rmrow, i) + t1row
                    rssp = _take(rsrow, i)
                    for hh in range(H):
                        wr = _take(erow, hh) * rssp
                        for j in range(NVD):
                            ps[hh][j] = wr * hv[j] + ps[hh][j]
                return (*ps[0], *ps[1], *ps[2], *ps[3], srow, t1row)

            init = (tuple(pacc[pl.ds(pb0 + hh * D + j * L, L)]
                          for hh in range(H) for j in range(NVD))
                    + (sacc[si], t1acc[si]))
            res = lax.fori_loop(0, NG, gloop, init)
            for hh in range(H):
                for j in range(NVD):
                    pacc[pl.ds(pb0 + hh * D + j * L, L)] = res[hh * NVD + j]
            sacc[si] = res[H * NVD]
            t1acc[si] = res[H * NVD + 1]

        @pl.when(jnp.logical_not(uniform))
        def _():
            @pl.loop(0, NG)
            def _(g):
                gt = ch * NG + g
                batchrow = batchbuf[pl.ds(gt * L, L)]
                murow = mubuf[pl.ds(gt * L, L)]
                rsrow = rsbuf[pl.ds(gt * L, L)]
                rmrow = rsrow * murow
                for i in range(L):
                    off = (g * L + i) * D
                    hv = [hbuf[pl.ds(off + j * L, L)] for j in range(NVD)]
                    seg = batchrow[i]
                    rmsp = _take(rmrow, i)
                    rssp = _take(rsrow, i)
                    lrow = logbuf[pl.ds((g * L + i) * L, L)]
                    mrow = mbuf[pl.ds(seg * L, L)]
                    erow = jnp.exp(lrow - mrow)
                    si = pl.ds(seg * L, L)
                    sacc[si] = sacc[si] + erow
                    t1acc[si] = erow * rmsp + t1acc[si]
                    for hh in range(H):
                        wr = _take(erow, hh) * rssp
                        pb = seg * (H * D) + hh * D
                        for j in range(NVD):
                            idx = pl.ds(pb + j * L, L)
                            pacc[idx] = wr * hv[j] + pacc[idx]

    @pl.loop(0, NCH, step=2)
    def _(ch):
        t1b = (tok0 + (ch + 1) * CW)
        pltpu.async_copy(h_hbm.at[pl.ds(t1b * D, CW * D)], hbufB, semB)
        pltpu.async_copy(log_hbm.at[pl.ds(t1b * L, CW * L)], logbufB, semB)
        pltpu.make_async_copy(h_hbm.at[pl.ds(0, CW * D)], hbufA, semA).wait()
        pltpu.make_async_copy(log_hbm.at[pl.ds(0, CW * L)], logbufA,
                              semA).wait()
        _chunk(ch, hbufA, logbufA)

        @pl.when(ch + 2 < NCH)
        def _():
            t2b = (tok0 + (ch + 2) * CW)
            pltpu.async_copy(h_hbm.at[pl.ds(t2b * D, CW * D)], hbufA, semA)
            pltpu.async_copy(log_hbm.at[pl.ds(t2b * L, CW * L)], logbufA,
                             semA)

        pltpu.make_async_copy(h_hbm.at[pl.ds(0, CW * D)], hbufB, semB).wait()
        pltpu.make_async_copy(log_hbm.at[pl.ds(0, CW * L)], logbufB,
                              semB).wait()
        _chunk(ch + 1, hbufB, logbufB)

    cpo = []
    for b in range(B):
        cpo.append(pltpu.async_copy(
            pacc.at[pl.ds(b * (H * D), H * D)],
            pp_hbm.at[pl.ds((b * NW + w) * (H * D), H * D)], sem))
        cpo.append(pltpu.async_copy(
            sacc.at[pl.ds(b * L, L)],
            sp_hbm.at[pl.ds((b * NW + w) * L, L)], sem))
        cpo.append(pltpu.async_copy(
            t1acc.at[pl.ds(b * L, L)],
            t1_hbm.at[pl.ds((b * NW + w) * L, L)], sem))
    for cp in cpo:
        cp.wait()


_k4 = functools.partial(
    pl.kernel,
    out_type=(
        jax.ShapeDtypeStruct((NW * B * H * D,), F32),
        jax.ShapeDtypeStruct((NW * B * L,), F32),
        jax.ShapeDtypeStruct((NW * B * L,), F32),
    ),
    mesh=_mesh,
    scratch_types=[
        pltpu.VMEM((CW * D,), F32),
        pltpu.VMEM((CW * D,), F32),
        pltpu.VMEM((TPW,), jnp.int32),
        pltpu.VMEM((TPW,), F32),
        pltpu.VMEM((TPW,), F32),
        pltpu.VMEM((CW * L,), F32),
        pltpu.VMEM((CW * L,), F32),
        pltpu.VMEM((NW * B * L,), F32),
        pltpu.VMEM((B * L,), F32),
        pltpu.VMEM((B * H * D,), F32),
        pltpu.VMEM((B * L,), F32),
        pltpu.VMEM((B * L,), F32),
        pltpu.SemaphoreType.DMA,
        pltpu.SemaphoreType.DMA,
        pltpu.SemaphoreType.DMA,
    ],
)(_k4_body)


def _k5_body(pp_hbm, sp_hbm, t1_hbm, mean_hbm, wv_hbm, woutt_hbm, wrest_hbm,
             lnkg_hbm, lnkb_hbm,
             out_hbm,
             pbuf, sbuf, t1buf, meanbuf, wvbuf, woutbuf, wresbuf,
             gkbuf, bkbuf, abuf, obuf, sem):
    w = _wid()
    b = w // 2
    o0 = (w % 2) * (D // 2)
    cps = [
        pltpu.async_copy(pp_hbm.at[pl.ds(b * (NW * H * D), NW * H * D)],
                         pbuf, sem),
        pltpu.async_copy(sp_hbm.at[pl.ds(b * (NW * L), NW * L)], sbuf, sem),
        pltpu.async_copy(t1_hbm.at[pl.ds(b * (NW * L), NW * L)], t1buf, sem),
        pltpu.async_copy(mean_hbm.at[pl.ds(b * D, D)], meanbuf, sem),
        pltpu.async_copy(wv_hbm, wvbuf, sem),
        pltpu.async_copy(woutt_hbm, woutbuf, sem),
        pltpu.async_copy(wrest_hbm, wresbuf, sem),
        pltpu.async_copy(lnkg_hbm, gkbuf, sem),
        pltpu.async_copy(lnkb_hbm, bkbuf, sem),
    ]
    for cp in cps:
        cp.wait()

    srow = sbuf[pl.ds(0, L)]
    t1row = t1buf[pl.ds(0, L)]
    for wi in range(1, NW):
        srow = srow + sbuf[pl.ds(wi * L, L)]
        t1row = t1row + t1buf[pl.ds(wi * L, L)]

    for hh in range(H):
        sv = _take(srow, hh)
        t1v = _take(t1row, hh)
        for j in range(NVD):
            pm = pbuf[pl.ds(hh * D + j * L, L)]
            for wi in range(1, NW):
                pm = pm + pbuf[pl.ds(wi * (H * D) + hh * D + j * L, L)]
            gk = gkbuf[pl.ds(j * L, L)]
            bk = bkbuf[pl.ds(j * L, L)]
            abuf[pl.ds(hh * D + j * L, L)] = (gk * (pm - t1v) + bk * sv) / sv

    zero4 = tuple(jnp.zeros((L,), F32) for _ in range(4))
    oacc = zero4
    for hh in range(H):
        av = [abuf[pl.ds(hh * D + j * L, L)] for j in range(NVD)]

        def kstep(k2, oaccs, hh=hh, av=av):
            k = hh * HD + k2
            p = av[0] * wvbuf[pl.ds(k * D, L)]
            for j in range(1, NVD):
                p = av[j] * wvbuf[pl.ds(k * D + j * L, L)] + p
            gs = _allsum(p)
            return tuple(oaccs[m] + gs * woutbuf[pl.ds(k * D + o0 + m * L, L)]
                         for m in range(4))

        oacc = lax.fori_loop(0, HD, kstep, oacc)

    def rstep(gk2, raccs):
        mrow = meanbuf[pl.ds(gk2 * L, L)]
        for i in range(L):
            ms = _take(mrow, i)
            k = gk2 * L + i
            raccs = tuple(
                raccs[m] + ms * wresbuf[pl.ds(k * D + o0 + m * L, L)]
                for m in range(4))
        return raccs

    racc = lax.fori_loop(0, D // L, rstep, zero4)
    for m in range(4):
        obuf[pl.ds(m * L, L)] = oacc[m] + np.float32(MRS) * racc[m]
    pltpu.sync_copy(obuf, out_hbm.at[pl.ds(b * D + o0, D // 2)])


_k5 = functools.partial(
    pl.kernel,
    out_type=jax.ShapeDtypeStruct((B * D,), F32),
    mesh=_mesh,
    scratch_types=[
        pltpu.VMEM((NW * H * D,), F32),
        pltpu.VMEM((NW * L,), F32),
        pltpu.VMEM((NW * L,), F32),
        pltpu.VMEM((D,), F32),
        pltpu.VMEM((D * D,), F32),
        pltpu.VMEM((D * D,), F32),
        pltpu.VMEM((D * D,), F32),
        pltpu.VMEM((D,), F32),
        pltpu.VMEM((D,), F32),
        pltpu.VMEM((H * D,), F32),
        pltpu.VMEM((D // 2,), F32),
        pltpu.SemaphoreType.DMA,
    ],
)(_k5_body)


def kernel(h, batch, cdr_mask, iface_mask, Wk, Wv, Wq, Wres, Wout,
           ln_kv_g, ln_kv_b, ln_q_g, ln_q_b, cdr_bias, iface_bias,
           logit_scale):
    h_flat = h.reshape(-1)
    cdrf = cdr_mask.astype(F32)
    ifacef = iface_mask.astype(F32)
    params = jnp.concatenate([
        jnp.stack([cdr_bias.astype(F32), iface_bias.astype(F32),
                   logit_scale.astype(F32)]),
        jnp.zeros((L - 3,), F32),
    ])
    wqt = Wq.T.reshape(-1)
    wk = Wk.reshape(-1)
    wv = Wv.reshape(-1)
    woutt = Wout.T.reshape(-1)
    wrest = Wres.T.reshape(-1)

    psum, pmax, pcnt, mu, rs = _k1(h_flat, batch)
    mean, qpg, s1c = _k2(psum, pmax, pcnt, wqt, wk,
                         ln_q_g, ln_q_b, ln_kv_g, ln_kv_b, params)
    logits, pmaxl = _k3(h_flat, batch, mu, rs, qpg, s1c, cdrf, ifacef, params)
    pp, sp, t1 = _k4(h_flat, batch, mu, rs, logits, pmaxl)
    out = _k5(pp, sp, t1, mean, wv, woutt, wrest, ln_kv_g, ln_kv_b)
    return out.reshape(B, D)
